# baseline (speedup 1.0000x reference)
import functools

import jax
import jax.numpy as jnp
from jax import lax
from jax.experimental import pallas as pl
from jax.experimental.pallas import tpu as pltpu
from jax.experimental.pallas import tpu_sc as plsc

NSEG = 512
N = 320000
D = 128
DV = D // 16

R0 = 86528

BLK = 512
NBLK = R0 // BLK

NW = 32
NSC = N - R0
ROWS_W = NSC // NW
C = 128
NCHUNK = ROWS_W // C
NS = 6
ZR = NSEG // 16


def _sc_body(x_hbm, b2d_hbm, out_hbm, xbuf, ids, zbuf, shared, sems, ssems):
    cid = lax.axis_index("c")
    sid = lax.axis_index("s")
    wid = sid * 2 + cid
    base = R0 + wid * ROWS_W

    def dma_x(k, s):
        return pltpu.make_async_copy(
            x_hbm.at[pl.ds(base + k * C, C)], xbuf.at[s], sems.at[s]
        )

    def scat(k, s):
        return pltpu.make_async_copy(
            xbuf.at[s], shared.at[ids.at[k]], ssems.at[s]
        )

    idcp = pltpu.make_async_copy(b2d_hbm.at[wid], ids, sems.at[NS])
    idcp.start()

    zero = jnp.zeros((16,), jnp.float32)

    def zrow(r, carry):
        row = zbuf.at[r]
        for j in range(DV):
            row[pl.ds(16 * j, 16)] = zero
        return carry

    lax.fori_loop(0, ZR, zrow, 0)
    pltpu.sync_copy(zbuf, shared.at[pl.ds(sid * ZR, ZR)])
    plsc.subcore_barrier()

    for s in range(NS):
        dma_x(s, s).start()
    idcp.wait()

    def ring(q, carry):
        for s in range(NS):
            k = NS * q + s
            dma_x(k, s).wait()
            scat(k, s).start(add=True)
        for s in range(NS):
            k = NS * q + s
            scat(k, s).wait()

            @pl.when(k + NS < NCHUNK)
            def _(k=k, s=s):
                dma_x(k + NS, s).start()

        return carry

    lax.fori_loop(0, NCHUNK // NS, ring, 0)
    for r in range(NCHUNK % NS):
        k = (NCHUNK // NS) * NS + r
        dma_x(k, r).wait()
        scat(k, r).start(add=True)
    for r in range(NCHUNK % NS):
        k = (NCHUNK // NS) * NS + r
        scat(k, r).wait()

    plsc.subcore_barrier()
    pltpu.sync_copy(
        shared.at[pl.ds(sid * ZR, ZR)],
        out_hbm.at[cid].at[pl.ds(sid * ZR, ZR)],
    )


def _tc_body(batch_ref, x_ref, out_ref):
    i = pl.program_id(0)
    b = batch_ref[0, 0, :]
    onehot = (
        jax.lax.broadcasted_iota(jnp.int32, (NSEG, BLK), 0) == b[None, :]
    ).astype(jnp.float32)
    part = jax.lax.dot_general(
        onehot, x_ref[...], (((1,), (0,)), ((), ())),
        preferred_element_type=jnp.float32,
    )

    @pl.when(i == 0)
    def _():
        out_ref[...] = part

    @pl.when(i > 0)
    def _():
        out_ref[...] += part


def _combine_body(t_ref, p_ref, o_ref):
    o_ref[...] = t_ref[...] + p_ref[0] + p_ref[1]


def kernel(x, batch):
    b32 = batch.astype(jnp.int32)
    b2d = b32[R0:].reshape(NW, NCHUNK, C)
    batch3 = b32.reshape(N // BLK, 1, BLK)

    sc = pl.kernel(
        _sc_body,
        out_type=jax.ShapeDtypeStruct((2, NSEG, D), jnp.float32),
        mesh=plsc.VectorSubcoreMesh(core_axis_name="c", subcore_axis_name="s"),
        scratch_types=[
            pltpu.VMEM((NS, C, D), jnp.float32),
            pltpu.VMEM((NCHUNK, C), jnp.int32),
            pltpu.VMEM((ZR, D), jnp.float32),
            pltpu.VMEM_SHARED((NSEG, D), jnp.float32),
            pltpu.SemaphoreType.DMA((NS + 1,)),
            pltpu.SemaphoreType.DMA((NS,)),
        ],
    )
    partials = sc(x, b2d)

    tcp = pl.pallas_call(
        _tc_body,
        grid=(NBLK,),
        in_specs=[
            pl.BlockSpec((1, 1, BLK), lambda i: (i, 0, 0)),
            pl.BlockSpec((BLK, D), lambda i: (i, 0)),
        ],
        out_specs=pl.BlockSpec((NSEG, D), lambda i: (0, 0)),
        out_shape=jax.ShapeDtypeStruct((NSEG, D), jnp.float32),
    )(batch3, x)

    out = pl.pallas_call(
        _combine_body,
        out_shape=jax.ShapeDtypeStruct((NSEG, D), jnp.float32),
    )(tcp, partials)
    return out


# speedup vs baseline: 1.5153x; 1.5153x over previous
import functools

import jax
import jax.numpy as jnp
from jax import lax
from jax.experimental import pallas as pl
from jax.experimental.pallas import tpu as pltpu
from jax.experimental.pallas import tpu_sc as plsc

NSEG = 512
N = 320000
D = 128
DV = D // 16

R0 = 94720

BLK = 1280
NBLK = R0 // BLK

NW = 32
NSC = N - R0
ROWS_W = NSC // NW
C = 128
NCHUNK = ROWS_W // C
NS = 6
ZR = NSEG // 16


def _sc_body(x_hbm, b2d_hbm, out_hbm, xbuf, ids, zbuf, shared, sems, ssems):
    cid = lax.axis_index("c")
    sid = lax.axis_index("s")
    wid = sid * 2 + cid
    base = R0 + wid * ROWS_W

    def dma_x(k, s):
        return pltpu.make_async_copy(
            x_hbm.at[pl.ds(base + k * C, C)], xbuf.at[s], sems.at[s]
        )

    def scat(k, s):
        return pltpu.make_async_copy(
            xbuf.at[s], shared.at[ids.at[k]], ssems.at[s]
        )

    idcp = pltpu.make_async_copy(b2d_hbm.at[wid], ids, sems.at[NS])
    idcp.start()

    zero = jnp.zeros((16,), jnp.float32)

    def zrow(r, carry):
        row = zbuf.at[r]
        for j in range(DV):
            row[pl.ds(16 * j, 16)] = zero
        return carry

    lax.fori_loop(0, ZR, zrow, 0)
    pltpu.sync_copy(zbuf, shared.at[pl.ds(sid * ZR, ZR)])
    plsc.subcore_barrier()

    for s in range(NS):
        dma_x(s, s).start()
    idcp.wait()

    def ring(q, carry):
        for s in range(NS):
            k = NS * q + s
            dma_x(k, s).wait()
            scat(k, s).start(add=True)
        for s in range(NS):
            k = NS * q + s
            scat(k, s).wait()

            @pl.when(k + NS < NCHUNK)
            def _(k=k, s=s):
                dma_x(k + NS, s).start()

        return carry

    lax.fori_loop(0, NCHUNK // NS, ring, 0)
    for r in range(NCHUNK % NS):
        k = (NCHUNK // NS) * NS + r
        dma_x(k, r).wait()
        scat(k, r).start(add=True)
    for r in range(NCHUNK % NS):
        k = (NCHUNK // NS) * NS + r
        scat(k, r).wait()

    plsc.subcore_barrier()
    pltpu.sync_copy(
        shared.at[pl.ds(sid * ZR, ZR)],
        out_hbm.at[cid].at[pl.ds(sid * ZR, ZR)],
    )


def _tc_body(batch_ref, x_ref, out_ref):
    i = pl.program_id(0)
    b = batch_ref[0, 0, :]
    onehot = (
        jax.lax.broadcasted_iota(jnp.int32, (NSEG, BLK), 0) == b[None, :]
    ).astype(jnp.float32)
    part = jax.lax.dot_general(
        onehot, x_ref[...], (((1,), (0,)), ((), ())),
        preferred_element_type=jnp.float32,
    )

    @pl.when(i == 0)
    def _():
        out_ref[...] = part

    @pl.when(i > 0)
    def _():
        out_ref[...] += part


def _combine_body(t_ref, p_ref, o_ref):
    o_ref[...] = t_ref[...] + p_ref[0] + p_ref[1]


def kernel(x, batch):
    b32 = batch.astype(jnp.int32)
    b2d = b32[R0:].reshape(NW, NCHUNK, C)
    batch3 = b32.reshape(N // BLK, 1, BLK)

    sc = pl.kernel(
        _sc_body,
        out_type=jax.ShapeDtypeStruct((2, NSEG, D), jnp.float32),
        mesh=plsc.VectorSubcoreMesh(core_axis_name="c", subcore_axis_name="s"),
        scratch_types=[
            pltpu.VMEM((NS, C, D), jnp.float32),
            pltpu.VMEM((NCHUNK, C), jnp.int32),
            pltpu.VMEM((ZR, D), jnp.float32),
            pltpu.VMEM_SHARED((NSEG, D), jnp.float32),
            pltpu.SemaphoreType.DMA((NS + 1,)),
            pltpu.SemaphoreType.DMA((NS,)),
        ],
    )
    partials = sc(x, b2d)

    tcp = pl.pallas_call(
        _tc_body,
        grid=(NBLK,),
        in_specs=[
            pl.BlockSpec((1, 1, BLK), lambda i: (i, 0, 0)),
            pl.BlockSpec((BLK, D), lambda i: (i, 0)),
        ],
        out_specs=pl.BlockSpec((NSEG, D), lambda i: (0, 0)),
        out_shape=jax.ShapeDtypeStruct((NSEG, D), jnp.float32),
    )(batch3, x)

    out = pl.pallas_call(
        _combine_body,
        out_shape=jax.ShapeDtypeStruct((NSEG, D), jnp.float32),
    )(tcp, partials)
    return out
